# baseline (speedup 1.0000x reference)
import functools

import jax
import jax.numpy as jnp
from jax import lax
from jax.experimental import pallas as pl
from jax.experimental.pallas import tpu as pltpu
from jax.experimental.pallas import tpu_sc as plsc

D_MODEL = 1024
SEQ = 4097
MAIN_ROWS = 4096
SPILL = 8
CHUNK = 16
NBUF = 6

_info = plsc.get_sparse_core_info()
_NC, _NS = _info.num_cores, _info.num_subcores
_NW = _NC * _NS
_B_PER_W = 128


def _gather_body(pix_hbm, cond_hbm, tok_hbm, out_hbm, spill_hbm,
                 idx_v, tidx_v, crow_v, bufs,
                 g0, g1, g2, g3, g4, g5, g6,
                 w0, w1, w2, w3, w4, w5, w6, csem):
    gsems = [g0, g1, g2, g3, g4, g5, g6]
    wsems = [w0, w1, w2, w3, w4, w5, w6]
    wid = lax.axis_index("s") * _NC + lax.axis_index("c")
    base = pl.multiple_of(wid * _B_PER_W, _B_PER_W)

    pltpu.sync_copy(tok_hbm.at[pl.ds(base, _B_PER_W)], idx_v)

    def run_ring(chunks, fix_row0, after_prime=None):
        def gath(i, b):
            ref, off, size, _, _ = chunks[i]
            return pltpu.make_async_copy(
                pix_hbm.at[ref.at[pl.ds(off, size)]],
                bufs.at[b].at[pl.ds(0, size)], gsems[b],
            )

        def wr(i, b):
            _, _, size, dst, doff = chunks[i]
            return pltpu.make_async_copy(
                bufs.at[b].at[pl.ds(0, size)],
                dst.at[pl.ds(doff, size)], wsems[b],
            )

        for i in range(min(NBUF, len(chunks))):
            gath(i, i).start()
        if after_prime is not None:
            after_prime()

        pending = {}
        for i in range(len(chunks)):
            b = i % NBUF
            gath(i, b).wait()
            if fix_row0 and i == 0:
                @pl.when(wid == 0)
                def _():
                    pltpu.make_async_copy(
                        cond_hbm.at[idx_v.at[pl.ds(0, 8)]], crow_v, csem
                    ).wait()
                    buf0 = bufs.at[b]
                    for j in range(D_MODEL // 16):
                        buf0[0, pl.ds(j * 16, 16)] = (
                            crow_v[0, pl.ds(j * 16, 16)]
                        )
            wr(i, b).start()
            pending[b] = i
            if i + NBUF < len(chunks):
                wr(i, b).wait()
                del pending[b]
                gath(i + NBUF, b).start()

        for b, i in pending.items():
            wr(i, b).wait()

    n_main = _B_PER_W // CHUNK
    main = [(idx_v, c * CHUNK, CHUNK, out_hbm, base + c * CHUNK)
            for c in range(n_main)]

    @pl.when(wid < _NW - 1)
    def _():
        def prefetch_cond():
            @pl.when(wid == 0)
            def _():
                pltpu.async_copy(
                    cond_hbm.at[idx_v.at[pl.ds(0, 8)]], crow_v, csem
                ).start()

        run_ring(main, fix_row0=True, after_prime=prefetch_cond)

    @pl.when(wid == _NW - 1)
    def _():
        pltpu.async_copy(tok_hbm.at[pl.ds(base + 120, 9)],
                         tidx_v.at[pl.ds(0, 9)], csem).start()

        def finish_tail_idx():
            pltpu.make_async_copy(tok_hbm.at[pl.ds(base + 120, 9)],
                                  tidx_v.at[pl.ds(0, 9)], csem).wait()
            v = tidx_v[pl.ds(0, 16)]
            lanes = lax.broadcasted_iota(jnp.int32, (16,), 0)
            tidx_v[pl.ds(0, 16)] = jnp.where(lanes == 8, v, 0)

        run_ring(main + [(tidx_v, 8, 8, spill_hbm, 0)], fix_row0=False,
                 after_prime=finish_tail_idx)


def _patch_body(main_ref, spill_ref, out_ref):
    del main_ref
    out_ref[...] = spill_ref[...]


@jax.jit
def _pixel_encoding(tokens, pixel_embed, cond_embed):
    mesh = plsc.VectorSubcoreMesh(core_axis_name="c", subcore_axis_name="s")
    run = functools.partial(
        pl.kernel,
        mesh=mesh,
        out_type=(
            jax.ShapeDtypeStruct((SEQ, D_MODEL), jnp.float32),
            jax.ShapeDtypeStruct((SPILL, D_MODEL), jnp.float32),
        ),
        scratch_types=[
            pltpu.VMEM((_B_PER_W,), jnp.int32),
            pltpu.VMEM((16,), jnp.int32),
            pltpu.VMEM((8, D_MODEL), jnp.float32),
            pltpu.VMEM((NBUF, CHUNK, D_MODEL), jnp.float32),
        ] + [pltpu.SemaphoreType.DMA] * 15,
        compiler_params=pltpu.CompilerParams(skip_device_barrier=True),
    )(_gather_body)
    main, spill = run(pixel_embed, cond_embed, tokens)

    return pl.pallas_call(
        _patch_body,
        out_shape=jax.ShapeDtypeStruct((SEQ, D_MODEL), jnp.float32),
        grid=(1,),
        in_specs=[
            pl.BlockSpec(memory_space=pl.ANY),
            pl.BlockSpec((8, D_MODEL), lambda i: (0, 0)),
        ],
        out_specs=pl.BlockSpec((8, D_MODEL), lambda i: (MAIN_ROWS // 8, 0)),
        input_output_aliases={0: 0},
        compiler_params=pltpu.CompilerParams(skip_device_barrier=True),
    )(main, spill)


def kernel(tokens, pixel_embed, cond_embed):
    return _pixel_encoding(tokens, pixel_embed, cond_embed)


# speedup vs baseline: 1.0020x; 1.0020x over previous
import functools

import jax
import jax.numpy as jnp
from jax import lax
from jax.experimental import pallas as pl
from jax.experimental.pallas import tpu as pltpu
from jax.experimental.pallas import tpu_sc as plsc

D_MODEL = 1024
SEQ = 4097
MAIN_ROWS = 4096
SPILL = 8
CHUNK = 16
NBUF = 6

_info = plsc.get_sparse_core_info()
_NC, _NS = _info.num_cores, _info.num_subcores
_NW = _NC * _NS
_B_PER_W = 128
_SPILL_W = 17


def _gather_body(pix_hbm, cond_hbm, tok_hbm, out_hbm, spill_hbm,
                 idx_v, tidx_v, crow_v, bufs,
                 g0, g1, g2, g3, g4, g5, g6,
                 w0, w1, w2, w3, w4, w5, w6, csem):
    gsems = [g0, g1, g2, g3, g4, g5, g6]
    wsems = [w0, w1, w2, w3, w4, w5, w6]
    wid = lax.axis_index("s") * _NC + lax.axis_index("c")
    base = pl.multiple_of(wid * _B_PER_W, _B_PER_W)

    pltpu.sync_copy(tok_hbm.at[pl.ds(base, _B_PER_W)], idx_v)

    def run_ring(chunks, fix_row0, after_prime=None):
        def gath(i, b):
            ref, off, size, _, _ = chunks[i]
            return pltpu.make_async_copy(
                pix_hbm.at[ref.at[pl.ds(off, size)]],
                bufs.at[b].at[pl.ds(0, size)], gsems[b],
            )

        def wr(i, b):
            _, _, size, dst, doff = chunks[i]
            return pltpu.make_async_copy(
                bufs.at[b].at[pl.ds(0, size)],
                dst.at[pl.ds(doff, size)], wsems[b],
            )

        for i in range(min(NBUF, len(chunks))):
            gath(i, i).start()
        if after_prime is not None:
            after_prime()

        pending = {}
        for i in range(len(chunks)):
            b = i % NBUF
            gath(i, b).wait()
            if fix_row0 and i == 0:
                @pl.when(wid == 0)
                def _():
                    pltpu.make_async_copy(
                        cond_hbm.at[idx_v.at[pl.ds(0, 8)]], crow_v, csem
                    ).wait()
                    buf0 = bufs.at[b]
                    for j in range(D_MODEL // 16):
                        buf0[0, pl.ds(j * 16, 16)] = (
                            crow_v[0, pl.ds(j * 16, 16)]
                        )
            wr(i, b).start()
            pending[b] = i
            if i + NBUF < len(chunks):
                wr(i, b).wait()
                del pending[b]
                gath(i + NBUF, b).start()

        for b, i in pending.items():
            wr(i, b).wait()

    n_main = _B_PER_W // CHUNK
    main = [(idx_v, c * CHUNK, CHUNK, out_hbm, base + c * CHUNK)
            for c in range(n_main)]

    @pl.when(wid != _SPILL_W)
    def _():
        def prefetch_cond():
            @pl.when(wid == 0)
            def _():
                pltpu.async_copy(
                    cond_hbm.at[idx_v.at[pl.ds(0, 8)]], crow_v, csem
                ).start()

        run_ring(main, fix_row0=True, after_prime=prefetch_cond)

    @pl.when(wid == _SPILL_W)
    def _():
        pltpu.async_copy(tok_hbm.at[pl.ds(SEQ - 9, 9)],
                         tidx_v.at[pl.ds(0, 9)], csem).start()

        def finish_tail_idx():
            pltpu.make_async_copy(tok_hbm.at[pl.ds(SEQ - 9, 9)],
                                  tidx_v.at[pl.ds(0, 9)], csem).wait()
            v = tidx_v[pl.ds(0, 16)]
            lanes = lax.broadcasted_iota(jnp.int32, (16,), 0)
            tidx_v[pl.ds(0, 16)] = jnp.where(lanes == 8, v, 0)

        run_ring(main + [(tidx_v, 8, 8, spill_hbm, 0)], fix_row0=False,
                 after_prime=finish_tail_idx)


def _patch_body(main_ref, spill_ref, out_ref):
    del main_ref
    out_ref[...] = spill_ref[...]


@jax.jit
def _pixel_encoding(tokens, pixel_embed, cond_embed):
    mesh = plsc.VectorSubcoreMesh(core_axis_name="c", subcore_axis_name="s")
    run = functools.partial(
        pl.kernel,
        mesh=mesh,
        out_type=(
            jax.ShapeDtypeStruct((SEQ, D_MODEL), jnp.float32),
            jax.ShapeDtypeStruct((SPILL, D_MODEL), jnp.float32),
        ),
        scratch_types=[
            pltpu.VMEM((_B_PER_W,), jnp.int32),
            pltpu.VMEM((16,), jnp.int32),
            pltpu.VMEM((8, D_MODEL), jnp.float32),
            pltpu.VMEM((NBUF, CHUNK, D_MODEL), jnp.float32),
        ] + [pltpu.SemaphoreType.DMA] * 15,
        compiler_params=pltpu.CompilerParams(skip_device_barrier=True),
    )(_gather_body)
    main, spill = run(pixel_embed, cond_embed, tokens)

    return pl.pallas_call(
        _patch_body,
        out_shape=jax.ShapeDtypeStruct((SEQ, D_MODEL), jnp.float32),
        grid=(1,),
        in_specs=[
            pl.BlockSpec(memory_space=pl.ANY),
            pl.BlockSpec((8, D_MODEL), lambda i: (0, 0)),
        ],
        out_specs=pl.BlockSpec((8, D_MODEL), lambda i: (MAIN_ROWS // 8, 0)),
        input_output_aliases={0: 0},
        compiler_params=pltpu.CompilerParams(skip_device_barrier=True),
    )(main, spill)


def kernel(tokens, pixel_embed, cond_embed):
    return _pixel_encoding(tokens, pixel_embed, cond_embed)


# speedup vs baseline: 1.0106x; 1.0086x over previous
import functools

import jax
import jax.numpy as jnp
from jax import lax
from jax.experimental import pallas as pl
from jax.experimental.pallas import tpu as pltpu
from jax.experimental.pallas import tpu_sc as plsc

D_MODEL = 1024
SEQ = 4097
CHUNK = 16
NBUF = 6

_info = plsc.get_sparse_core_info()
_NC, _NS = _info.num_cores, _info.num_subcores
_NW = _NC * _NS
_B_PER_W = 128
_COND_W = 0
_SPILL_W = 17


def _gather_body(pix_hbm, cond_hbm, tok_hbm, out_hbm, spill_hbm,
                 idx_v, tidx_v, crow_v, sbuf_v, bufs,
                 g0, g1, g2, g3, g4, g5, w0, w1, w2, w3, w4, w5, csem):
    gsems = [g0, g1, g2, g3, g4, g5]
    wsems = [w0, w1, w2, w3, w4, w5]
    wid = lax.axis_index("s") * _NC + lax.axis_index("c")
    base = pl.multiple_of(wid * _B_PER_W, _B_PER_W)

    pltpu.sync_copy(tok_hbm.at[pl.ds(base, _B_PER_W)], idx_v)

    n_chunks = _B_PER_W // CHUNK

    def gath(i, b):
        return pltpu.make_async_copy(
            pix_hbm.at[idx_v.at[pl.ds(i * CHUNK, CHUNK)]],
            bufs.at[b], gsems[b],
        )

    def wr(i, b):
        return pltpu.make_async_copy(
            bufs.at[b], out_hbm.at[pl.ds(base + i * CHUNK, CHUNK)], wsems[b],
        )

    for i in range(NBUF):
        gath(i, i).start()

    @pl.when(wid == _COND_W)
    def _():
        pltpu.async_copy(
            cond_hbm.at[idx_v.at[pl.ds(0, 8)]], crow_v, csem
        ).start()

    @pl.when(wid == _SPILL_W)
    def _():
        pltpu.sync_copy(tok_hbm.at[pl.ds(SEQ - 9, 8)],
                        tidx_v.at[pl.ds(8, 8)])
        pltpu.sync_copy(tok_hbm.at[pl.ds(SEQ - 9, 9)],
                        tidx_v.at[pl.ds(0, 9)])
        pltpu.async_copy(pix_hbm.at[tidx_v], sbuf_v, csem).start()

    pending = {}
    for i in range(n_chunks):
        b = i % NBUF
        gath(i, b).wait()
        wr(i, b).start()
        pending[b] = i
        if i + NBUF < n_chunks:
            wr(i, b).wait()
            del pending[b]
            gath(i + NBUF, b).start()

    for b, i in pending.items():
        wr(i, b).wait()

    @pl.when(wid == _COND_W)
    def _():
        pltpu.make_async_copy(
            cond_hbm.at[idx_v.at[pl.ds(0, 8)]], crow_v, csem
        ).wait()
        pltpu.sync_copy(crow_v, spill_hbm.at[pl.ds(0, 8)])

    @pl.when(wid == _SPILL_W)
    def _():
        pltpu.make_async_copy(pix_hbm.at[tidx_v], sbuf_v, csem).wait()
        pltpu.sync_copy(sbuf_v.at[pl.ds(8, 8)], spill_hbm.at[pl.ds(8, 8)])


def _patch_body(main_ref, spill_ref, out_ref):
    i = pl.program_id(0)
    rows = lax.broadcasted_iota(jnp.int32, (8, D_MODEL), 0)
    spill_row0 = jnp.broadcast_to(spill_ref[0:1, :], (8, D_MODEL))
    merged0 = jnp.where(rows == 0, spill_row0, main_ref[...])
    out_ref[...] = jnp.where(i == 0, merged0, spill_row0)


@jax.jit
def _pixel_encoding(tokens, pixel_embed, cond_embed):
    mesh = plsc.VectorSubcoreMesh(core_axis_name="c", subcore_axis_name="s")
    run = functools.partial(
        pl.kernel,
        mesh=mesh,
        out_type=(
            jax.ShapeDtypeStruct((SEQ, D_MODEL), jnp.float32),
            jax.ShapeDtypeStruct((16, D_MODEL), jnp.float32),
        ),
        scratch_types=[
            pltpu.VMEM((_B_PER_W,), jnp.int32),
            pltpu.VMEM((16,), jnp.int32),
            pltpu.VMEM((8, D_MODEL), jnp.float32),
            pltpu.VMEM((16, D_MODEL), jnp.float32),
            pltpu.VMEM((NBUF, CHUNK, D_MODEL), jnp.float32),
        ] + [pltpu.SemaphoreType.DMA] * 13,
        compiler_params=pltpu.CompilerParams(skip_device_barrier=True),
    )(_gather_body)
    main, spill = run(pixel_embed, cond_embed, tokens)

    return pl.pallas_call(
        _patch_body,
        out_shape=jax.ShapeDtypeStruct((SEQ, D_MODEL), jnp.float32),
        grid=(2,),
        in_specs=[
            pl.BlockSpec((8, D_MODEL), lambda i: (i * (SEQ // 8), 0)),
            pl.BlockSpec((8, D_MODEL), lambda i: (i, 0)),
        ],
        out_specs=pl.BlockSpec((8, D_MODEL), lambda i: (i * (SEQ // 8), 0)),
        input_output_aliases={0: 0},
        compiler_params=pltpu.CompilerParams(skip_device_barrier=True),
    )(main, spill)


def kernel(tokens, pixel_embed, cond_embed):
    return _pixel_encoding(tokens, pixel_embed, cond_embed)


# speedup vs baseline: 1.0123x; 1.0017x over previous
import functools

import jax
import jax.numpy as jnp
from jax import lax
from jax.experimental import pallas as pl
from jax.experimental.pallas import tpu as pltpu
from jax.experimental.pallas import tpu_sc as plsc

D_MODEL = 1024
SEQ = 4097
CHUNK = 16
NBUF = 6

_info = plsc.get_sparse_core_info()
_NC, _NS = _info.num_cores, _info.num_subcores
_NW = _NC * _NS
_B_PER_W = 128
_COND_W = 0
_SPILL_W = 17


def _gather_body(pix_hbm, cond_hbm, tok_hbm, out_hbm, spill_hbm,
                 idx_v, tidx_v, crow_v, sbuf_v, bufs,
                 g0, g1, g2, g3, g4, g5, w0, w1, w2, w3, w4, w5, csem):
    gsems = [g0, g1, g2, g3, g4, g5]
    wsems = [w0, w1, w2, w3, w4, w5]
    wid = lax.axis_index("s") * _NC + lax.axis_index("c")
    base = pl.multiple_of(wid * _B_PER_W, _B_PER_W)

    pltpu.sync_copy(tok_hbm.at[pl.ds(base, _B_PER_W)], idx_v)

    n_chunks = _B_PER_W // CHUNK

    def gath(i, b):
        return pltpu.make_async_copy(
            pix_hbm.at[idx_v.at[pl.ds(i * CHUNK, CHUNK)]],
            bufs.at[b], gsems[b],
        )

    def wr(i, b):
        return pltpu.make_async_copy(
            bufs.at[b], out_hbm.at[pl.ds(base + i * CHUNK, CHUNK)], wsems[b],
        )

    for i in range(NBUF):
        gath(i, i).start()

    @pl.when(wid == _COND_W)
    def _():
        pltpu.async_copy(
            cond_hbm.at[idx_v.at[pl.ds(0, 8)]], crow_v, csem
        ).start()

    @pl.when(wid == _SPILL_W)
    def _():
        pltpu.sync_copy(tok_hbm.at[pl.ds(SEQ - 9, 8)],
                        tidx_v.at[pl.ds(8, 8)])
        pltpu.sync_copy(tok_hbm.at[pl.ds(SEQ - 9, 9)],
                        tidx_v.at[pl.ds(0, 9)])
        pltpu.async_copy(pix_hbm.at[tidx_v], sbuf_v, csem).start()

    pending = {}
    for i in range(n_chunks):
        b = i % NBUF
        gath(i, b).wait()
        wr(i, b).start()
        pending[b] = i
        if i + NBUF < n_chunks:
            wr(i, b).wait()
            del pending[b]
            gath(i + NBUF, b).start()

    for b, i in pending.items():
        wr(i, b).wait()

    @pl.when(wid == _COND_W)
    def _():
        pltpu.make_async_copy(
            cond_hbm.at[idx_v.at[pl.ds(0, 8)]], crow_v, csem
        ).wait()
        pltpu.sync_copy(crow_v, spill_hbm.at[pl.ds(0, 8)])

    @pl.when(wid == _SPILL_W)
    def _():
        pltpu.make_async_copy(pix_hbm.at[tidx_v], sbuf_v, csem).wait()
        pltpu.sync_copy(sbuf_v.at[pl.ds(8, 8)], spill_hbm.at[pl.ds(8, 8)])


def _patch_body(main_ref, spill_ref, out_ref):
    i = pl.program_id(0)
    rows = lax.broadcasted_iota(jnp.int32, (8, D_MODEL), 0)
    spill_row0 = jnp.broadcast_to(spill_ref[0:1, :], (8, D_MODEL))
    merged0 = jnp.where(rows == 0, spill_row0, main_ref[...])
    out_ref[...] = jnp.where(i == 0, merged0, spill_row0)


@jax.jit
def _pixel_encoding(tokens, pixel_embed, cond_embed):
    mesh = plsc.VectorSubcoreMesh(core_axis_name="c", subcore_axis_name="s")
    run = functools.partial(
        pl.kernel,
        mesh=mesh,
        out_type=(
            jax.ShapeDtypeStruct((SEQ, D_MODEL), jnp.float32),
            jax.ShapeDtypeStruct((16, D_MODEL), jnp.float32),
        ),
        scratch_types=[
            pltpu.VMEM((_B_PER_W,), jnp.int32),
            pltpu.VMEM((16,), jnp.int32),
            pltpu.VMEM((8, D_MODEL), jnp.float32),
            pltpu.VMEM((16, D_MODEL), jnp.float32),
            pltpu.VMEM((NBUF, CHUNK, D_MODEL), jnp.float32),
        ] + [pltpu.SemaphoreType.DMA] * 13,
    )(_gather_body)
    main, spill = run(pixel_embed, cond_embed, tokens)

    return pl.pallas_call(
        _patch_body,
        out_shape=jax.ShapeDtypeStruct((SEQ, D_MODEL), jnp.float32),
        grid=(2,),
        in_specs=[
            pl.BlockSpec((8, D_MODEL), lambda i: (i * (SEQ // 8), 0)),
            pl.BlockSpec((8, D_MODEL), lambda i: (i, 0)),
        ],
        out_specs=pl.BlockSpec((8, D_MODEL), lambda i: (i * (SEQ // 8), 0)),
        input_output_aliases={0: 0},
    )(main, spill)


def kernel(tokens, pixel_embed, cond_embed):
    return _pixel_encoding(tokens, pixel_embed, cond_embed)
